# Initial kernel scaffold; baseline (speedup 1.0000x reference)
import jax
import jax.numpy as jnp
from jax.experimental import pallas as pl


def kernel(embeddings, adjacency_matrix, W0, a_src0, a_dst0, W1, a_src1, a_dst1):
    raise NotImplementedError("write your pallas kernel here")


# speedup vs baseline: 2.1719x; 2.1719x over previous
import functools

import jax
import jax.numpy as jnp
from jax.experimental import pallas as pl
from jax.experimental.pallas import tpu as pltpu

_N = 10000
_D = 128
_BLK = 1024
_NP = 10240
_NB = _NP // _BLK
_NEG = -1e9
_MINIT = -3.0e38


def _proj_kernel(x_ref, w_ref, asrc_ref, adst_ref, h_ref, s_ref, d_ref):
    r = pl.program_id(0)
    x = x_ref[...]
    h = jnp.dot(x, w_ref[...], preferred_element_type=jnp.float32)
    rows = r * _BLK + jax.lax.broadcasted_iota(jnp.int32, (_BLK, 1), 0)
    h = jnp.where(rows < _N, h, 0.0)
    h_ref[...] = h
    s_ref[...] = jnp.sum(h * asrc_ref[...], axis=1, keepdims=True)
    d = jnp.dot(adst_ref[...], h.T, preferred_element_type=jnp.float32)
    cols = r * _BLK + jax.lax.broadcasted_iota(jnp.int32, (1, _BLK), 1)
    d_ref[...] = jnp.where(cols < _N, d, _NEG)


def _flash_kernel(adj_ref, s_ref, d_ref, h_ref, out_ref,
                  acc_ref, m_ref, l_ref):
    j = pl.program_id(1)

    @pl.when(j == 0)
    def _init():
        m_ref[...] = jnp.full((_BLK, 1), _MINIT, jnp.float32)
        l_ref[...] = jnp.zeros((_BLK, 1), jnp.float32)
        acc_ref[...] = jnp.zeros((_BLK, _D), jnp.float32)

    e = s_ref[...] + d_ref[...]
    e = jnp.maximum(e, 0.2 * e)
    e = jnp.where(adj_ref[...] > 0, e, _NEG)
    m_prev = m_ref[...]
    m_new = jnp.maximum(m_prev, jnp.max(e, axis=1, keepdims=True))
    alpha = jnp.exp(m_prev - m_new)
    p = jnp.exp(e - m_new)
    l_ref[...] = l_ref[...] * alpha + jnp.sum(p, axis=1, keepdims=True)
    m_ref[...] = m_new
    pv = jnp.dot(p.astype(jnp.bfloat16), h_ref[...].astype(jnp.bfloat16),
                 preferred_element_type=jnp.float32)
    acc_ref[...] = acc_ref[...] * alpha + pv

    @pl.when(j == pl.num_programs(1) - 1)
    def _fin():
        l = l_ref[...]
        a = acc_ref[...] / jnp.where(l > 0, l, 1.0)
        out_ref[...] = jnp.where(a > 0, a, jnp.exp(a) - 1.0)


def _gat_layer(x, adj, w, a_src, a_dst, interpret=False):
    h, s, d = pl.pallas_call(
        _proj_kernel,
        grid=(_NB,),
        in_specs=[
            pl.BlockSpec((_BLK, _D), lambda r: (r, 0)),
            pl.BlockSpec((_D, _D), lambda r: (0, 0)),
            pl.BlockSpec((1, _D), lambda r: (0, 0)),
            pl.BlockSpec((1, _D), lambda r: (0, 0)),
        ],
        out_specs=[
            pl.BlockSpec((_BLK, _D), lambda r: (r, 0)),
            pl.BlockSpec((_BLK, 1), lambda r: (r, 0)),
            pl.BlockSpec((1, _BLK), lambda r: (0, r)),
        ],
        out_shape=[
            jax.ShapeDtypeStruct((_NP, _D), jnp.float32),
            jax.ShapeDtypeStruct((_NP, 1), jnp.float32),
            jax.ShapeDtypeStruct((1, _NP), jnp.float32),
        ],
        interpret=interpret,
    )(x, w, a_src.reshape(1, _D), a_dst.reshape(1, _D))

    out = pl.pallas_call(
        _flash_kernel,
        grid=(_NB, _NB),
        in_specs=[
            pl.BlockSpec((_BLK, _BLK), lambda i, j: (i, j)),
            pl.BlockSpec((_BLK, 1), lambda i, j: (i, 0)),
            pl.BlockSpec((1, _BLK), lambda i, j: (0, j)),
            pl.BlockSpec((_BLK, _D), lambda i, j: (j, 0)),
        ],
        out_specs=pl.BlockSpec((_BLK, _D), lambda i, j: (i, 0)),
        out_shape=jax.ShapeDtypeStruct((_NP, _D), jnp.float32),
        scratch_shapes=[
            pltpu.VMEM((_BLK, _D), jnp.float32),
            pltpu.VMEM((_BLK, 1), jnp.float32),
            pltpu.VMEM((_BLK, 1), jnp.float32),
        ],
        interpret=interpret,
    )(adj, s, d, h)
    return out


def kernel(embeddings, adjacency_matrix, W0, a_src0, a_dst0,
           W1, a_src1, a_dst1, interpret=False):
    x = _gat_layer(embeddings, adjacency_matrix, W0, a_src0, a_dst0,
                   interpret=interpret)
    x = _gat_layer(x, adjacency_matrix, W1, a_src1, a_dst1,
                   interpret=interpret)
    return x[:_N]


# speedup vs baseline: 2.6623x; 1.2258x over previous
import functools

import jax
import jax.numpy as jnp
from jax.experimental import pallas as pl
from jax.experimental.pallas import tpu as pltpu

_N = 10000
_D = 128
_BLK = 1024
_NP = 10240
_NB = _NP // _BLK
_NEG = -1e9
_LOG2E = 1.4426950408889634


def _proj_kernel(x_ref, w_ref, asrc_ref, adst_ref,
                 hext_ref, s_ref, d_ref, dmax_ref):
    r = pl.program_id(0)
    x = x_ref[...]
    h = jnp.dot(x, w_ref[...], preferred_element_type=jnp.float32)
    rows = r * _BLK + jax.lax.broadcasted_iota(jnp.int32, (_BLK, 1), 0)
    h = jnp.where(rows < _N, h, 0.0)
    ones = (jax.lax.broadcasted_iota(jnp.int32, (_BLK, _D), 1) == 0)
    hext = jnp.concatenate([h, ones.astype(jnp.float32)], axis=1)
    hext_ref[...] = hext.astype(jnp.bfloat16)
    s_ref[...] = jnp.sum(h * asrc_ref[...], axis=1, keepdims=True) * _LOG2E
    d = jnp.dot(adst_ref[...], h.T,
                preferred_element_type=jnp.float32) * _LOG2E
    cols = r * _BLK + jax.lax.broadcasted_iota(jnp.int32, (1, _BLK), 1)
    d = jnp.where(cols < _N, d, _NEG)
    d_ref[...] = d
    bmax = jnp.max(d, axis=1, keepdims=True)

    @pl.when(r == 0)
    def _init():
        dmax_ref[...] = jnp.full((1, 1), -3.0e38, jnp.float32)

    dmax_ref[...] = jnp.maximum(dmax_ref[...], bmax)


def _flash_kernel(adj_ref, s_ref, d_ref, dmax_ref, hext_ref, out_ref,
                  acc_ref, sm_ref, r2_ref):
    j = pl.program_id(1)

    @pl.when(j == 0)
    def _init():
        s = s_ref[...]
        emax = s + dmax_ref[...]
        m = jnp.maximum(emax, 0.2 * emax)
        sm_ref[...] = s - m
        r2_ref[...] = 0.2 * s - m
        acc_ref[...] = jnp.zeros((_BLK, 2 * _D), jnp.float32)

    dj = d_ref[...]
    t1 = sm_ref[...] + dj
    t2 = r2_ref[...] + 0.2 * dj
    z = jnp.maximum(t1, t2)
    z = jnp.where(adj_ref[...] > 0, z, _NEG)
    p = jnp.exp2(z)
    pv = jnp.dot(p.astype(jnp.bfloat16), hext_ref[...],
                 preferred_element_type=jnp.float32)
    acc_ref[...] += pv

    @pl.when(j == pl.num_programs(1) - 1)
    def _fin():
        acc = acc_ref[...]
        l = acc[:, _D:_D + 1]
        a = acc[:, :_D] / jnp.where(l > 0, l, 1.0)
        out_ref[...] = jnp.where(a > 0, a, jnp.exp(a) - 1.0)


def _gat_layer(x, adj, w, a_src, a_dst, interpret=False):
    hext, s, d, dmax = pl.pallas_call(
        _proj_kernel,
        grid=(_NB,),
        in_specs=[
            pl.BlockSpec((_BLK, _D), lambda r: (r, 0)),
            pl.BlockSpec((_D, _D), lambda r: (0, 0)),
            pl.BlockSpec((1, _D), lambda r: (0, 0)),
            pl.BlockSpec((1, _D), lambda r: (0, 0)),
        ],
        out_specs=[
            pl.BlockSpec((_BLK, 2 * _D), lambda r: (r, 0)),
            pl.BlockSpec((_BLK, 1), lambda r: (r, 0)),
            pl.BlockSpec((1, _BLK), lambda r: (0, r)),
            pl.BlockSpec((1, 1), lambda r: (0, 0)),
        ],
        out_shape=[
            jax.ShapeDtypeStruct((_NP, 2 * _D), jnp.bfloat16),
            jax.ShapeDtypeStruct((_NP, 1), jnp.float32),
            jax.ShapeDtypeStruct((1, _NP), jnp.float32),
            jax.ShapeDtypeStruct((1, 1), jnp.float32),
        ],
        interpret=interpret,
    )(x, w, a_src.reshape(1, _D), a_dst.reshape(1, _D))

    out = pl.pallas_call(
        _flash_kernel,
        grid=(_NB, _NB),
        in_specs=[
            pl.BlockSpec((_BLK, _BLK), lambda i, j: (i, j)),
            pl.BlockSpec((_BLK, 1), lambda i, j: (i, 0)),
            pl.BlockSpec((1, _BLK), lambda i, j: (0, j)),
            pl.BlockSpec((1, 1), lambda i, j: (0, 0)),
            pl.BlockSpec((_BLK, 2 * _D), lambda i, j: (j, 0)),
        ],
        out_specs=pl.BlockSpec((_BLK, _D), lambda i, j: (i, 0)),
        out_shape=jax.ShapeDtypeStruct((_NP, _D), jnp.float32),
        scratch_shapes=[
            pltpu.VMEM((_BLK, 2 * _D), jnp.float32),
            pltpu.VMEM((_BLK, 1), jnp.float32),
            pltpu.VMEM((_BLK, 1), jnp.float32),
        ],
        interpret=interpret,
    )(adj, s, d, dmax, hext)
    return out


def kernel(embeddings, adjacency_matrix, W0, a_src0, a_dst0,
           W1, a_src1, a_dst1, interpret=False):
    x = _gat_layer(embeddings, adjacency_matrix, W0, a_src0, a_dst0,
                   interpret=interpret)
    x = _gat_layer(x, adjacency_matrix, W1, a_src1, a_dst1,
                   interpret=interpret)
    return x[:_N]


# speedup vs baseline: 2.8002x; 1.0518x over previous
import functools

import jax
import jax.numpy as jnp
from jax.experimental import pallas as pl
from jax.experimental.pallas import tpu as pltpu

_N = 10000
_D = 128
_BLK = 1024
_NP = 10240
_NB = _NP // _BLK
_NEG = -1e9
_LOG2E = 1.4426950408889634


def _proj_kernel(x_ref, w_ref, asrc_ref, adst_ref,
                 hext_ref, s_ref, d_ref, dmax_ref):
    r = pl.program_id(0)
    x = x_ref[...]
    h = jnp.dot(x, w_ref[...], preferred_element_type=jnp.float32)
    rows = r * _BLK + jax.lax.broadcasted_iota(jnp.int32, (_BLK, 1), 0)
    h = jnp.where(rows < _N, h, 0.0)
    ones = (jax.lax.broadcasted_iota(jnp.int32, (_BLK, _D), 1) == 0)
    hext = jnp.concatenate([h, ones.astype(jnp.float32)], axis=1)
    hext_ref[...] = hext.astype(jnp.bfloat16)
    s_ref[...] = jnp.sum(h * asrc_ref[...], axis=1, keepdims=True) * _LOG2E
    d = jnp.dot(adst_ref[...], h.T,
                preferred_element_type=jnp.float32) * _LOG2E
    cols = r * _BLK + jax.lax.broadcasted_iota(jnp.int32, (1, _BLK), 1)
    d = jnp.where(cols < _N, d, _NEG)
    d_ref[...] = d
    bmax = jnp.max(d, axis=1, keepdims=True)

    @pl.when(r == 0)
    def _init():
        dmax_ref[...] = jnp.full((1, 1), -3.0e38, jnp.float32)

    dmax_ref[...] = jnp.maximum(dmax_ref[...], bmax)


def _flash_kernel(emit_mask, adj_ref, s_ref, d_ref, dmax_ref, hext_ref,
                  *refs):
    if emit_mask:
        out_ref, mask_ref, acc_ref, sm_ref, r2_ref = refs
    else:
        out_ref, acc_ref, sm_ref, r2_ref = refs
    j = pl.program_id(1)

    @pl.when(j == 0)
    def _init():
        s = s_ref[...]
        emax = s + dmax_ref[...]
        m = jnp.maximum(emax, 0.2 * emax)
        sm_ref[...] = s - m
        r2_ref[...] = 0.2 * s - m
        acc_ref[...] = jnp.zeros((_BLK, 2 * _D), jnp.float32)

    dj = d_ref[...]
    t1 = sm_ref[...] + dj
    t2 = r2_ref[...] + 0.2 * dj
    z = jnp.maximum(t1, t2)
    p = jnp.exp2(z).astype(jnp.bfloat16)
    if emit_mask:
        adjpos = adj_ref[...] > 0
        mask_ref[...] = adjpos.astype(jnp.int8)
        p = jnp.where(adjpos, p, jnp.bfloat16(0))
    else:
        p = p * adj_ref[...].astype(jnp.bfloat16)
    pv = jnp.dot(p, hext_ref[...], preferred_element_type=jnp.float32)
    acc_ref[...] += pv

    @pl.when(j == pl.num_programs(1) - 1)
    def _fin():
        acc = acc_ref[...]
        l = acc[:, _D:_D + 1]
        a = acc[:, :_D] / jnp.where(l > 0, l, 1.0)
        out_ref[...] = jnp.where(a > 0, a, jnp.exp(a) - 1.0)


def _gat_layer(x, adj, w, a_src, a_dst, emit_mask, interpret=False):
    hext, s, d, dmax = pl.pallas_call(
        _proj_kernel,
        grid=(_NB,),
        in_specs=[
            pl.BlockSpec((_BLK, _D), lambda r: (r, 0)),
            pl.BlockSpec((_D, _D), lambda r: (0, 0)),
            pl.BlockSpec((1, _D), lambda r: (0, 0)),
            pl.BlockSpec((1, _D), lambda r: (0, 0)),
        ],
        out_specs=[
            pl.BlockSpec((_BLK, 2 * _D), lambda r: (r, 0)),
            pl.BlockSpec((_BLK, 1), lambda r: (r, 0)),
            pl.BlockSpec((1, _BLK), lambda r: (0, r)),
            pl.BlockSpec((1, 1), lambda r: (0, 0)),
        ],
        out_shape=[
            jax.ShapeDtypeStruct((_NP, 2 * _D), jnp.bfloat16),
            jax.ShapeDtypeStruct((_NP, 1), jnp.float32),
            jax.ShapeDtypeStruct((1, _NP), jnp.float32),
            jax.ShapeDtypeStruct((1, 1), jnp.float32),
        ],
        interpret=interpret,
    )(x, w, a_src.reshape(1, _D), a_dst.reshape(1, _D))

    out_specs = [pl.BlockSpec((_BLK, _D), lambda i, j: (i, 0))]
    out_shape = [jax.ShapeDtypeStruct((_NP, _D), jnp.float32)]
    if emit_mask:
        out_specs.append(pl.BlockSpec((_BLK, _BLK), lambda i, j: (i, j)))
        out_shape.append(jax.ShapeDtypeStruct((_N, _N), jnp.int8))
    outs = pl.pallas_call(
        functools.partial(_flash_kernel, emit_mask),
        grid=(_NB, _NB),
        in_specs=[
            pl.BlockSpec((_BLK, _BLK), lambda i, j: (i, j)),
            pl.BlockSpec((_BLK, 1), lambda i, j: (i, 0)),
            pl.BlockSpec((1, _BLK), lambda i, j: (0, j)),
            pl.BlockSpec((1, 1), lambda i, j: (0, 0)),
            pl.BlockSpec((_BLK, 2 * _D), lambda i, j: (j, 0)),
        ],
        out_specs=out_specs,
        out_shape=out_shape,
        scratch_shapes=[
            pltpu.VMEM((_BLK, 2 * _D), jnp.float32),
            pltpu.VMEM((_BLK, 1), jnp.float32),
            pltpu.VMEM((_BLK, 1), jnp.float32),
        ],
        interpret=interpret,
    )(adj, s, d, dmax, hext)
    return outs


def kernel(embeddings, adjacency_matrix, W0, a_src0, a_dst0,
           W1, a_src1, a_dst1, interpret=False):
    x, mask8 = _gat_layer(embeddings, adjacency_matrix, W0, a_src0, a_dst0,
                          emit_mask=True, interpret=interpret)
    x, = _gat_layer(x, mask8, W1, a_src1, a_dst1,
                    emit_mask=False, interpret=interpret)
    return x[:_N]


# speedup vs baseline: 3.0185x; 1.0780x over previous
import functools

import jax
import jax.numpy as jnp
from jax.experimental import pallas as pl
from jax.experimental.pallas import tpu as pltpu

_N = 10000
_D = 128
_BLK = 1024
_NP = 10240
_NB = _NP // _BLK
_NEG = -1e9
_LOG2E = 1.4426950408889634


def _proj_kernel(x_ref, w_ref, asrc_ref, adst_ref,
                 hext_ref, e_ref, b_ref, dd_ref):
    r = pl.program_id(0)
    x = x_ref[...]
    h = jnp.dot(x, w_ref[...], preferred_element_type=jnp.float32)
    rows = r * _BLK + jax.lax.broadcasted_iota(jnp.int32, (_BLK, 1), 0)
    h = jnp.where(rows < _N, h, 0.0)
    ones = (jax.lax.broadcasted_iota(jnp.int32, (_BLK, _D), 1) == 0)
    hext = jnp.concatenate([h, ones.astype(jnp.float32)], axis=1)
    hext_ref[...] = hext.astype(jnp.bfloat16)
    s = jnp.sum(h * asrc_ref[...], axis=1, keepdims=True) * _LOG2E
    e_ref[...] = jnp.exp2(-0.8 * s)
    d = jnp.dot(adst_ref[...], h.T,
                preferred_element_type=jnp.float32) * _LOG2E
    cols = r * _BLK + jax.lax.broadcasted_iota(jnp.int32, (1, _BLK), 1)
    d = jnp.where(cols < _N, d, _NEG)
    b_ref[...] = jnp.exp2(d)
    dd_ref[...] = jnp.exp2(0.2 * d)


def _flash_kernel(emit_mask, adj_ref, e_ref, b_ref, dd_ref, hext_ref,
                  *refs):
    if emit_mask:
        out_ref, mask_ref, acc_ref = refs
    else:
        out_ref, acc_ref = refs
    j = pl.program_id(1)

    @pl.when(j == 0)
    def _init():
        acc_ref[...] = jnp.zeros((_BLK, 2 * _D), jnp.float32)

    p = jnp.maximum(b_ref[...], e_ref[...] * dd_ref[...])
    pb = p.astype(jnp.bfloat16)
    if emit_mask:
        adjpos = adj_ref[...] > 0
        mask_ref[...] = adjpos.astype(jnp.int8)
        pb = jnp.where(adjpos, pb, jnp.bfloat16(0))
    else:
        pb = pb * adj_ref[...].astype(jnp.bfloat16)
    pv = jnp.dot(pb, hext_ref[...], preferred_element_type=jnp.float32)
    acc_ref[...] += pv

    @pl.when(j == pl.num_programs(1) - 1)
    def _fin():
        acc = acc_ref[...]
        l = acc[:, _D:_D + 1]
        a = acc[:, :_D] / jnp.where(l > 0, l, 1.0)
        out_ref[...] = jnp.where(a > 0, a, jnp.exp(a) - 1.0)


def _gat_layer(x, adj, w, a_src, a_dst, emit_mask, interpret=False):
    hext, ee, bb, dd = pl.pallas_call(
        _proj_kernel,
        grid=(_NB,),
        in_specs=[
            pl.BlockSpec((_BLK, _D), lambda r: (r, 0)),
            pl.BlockSpec((_D, _D), lambda r: (0, 0)),
            pl.BlockSpec((1, _D), lambda r: (0, 0)),
            pl.BlockSpec((1, _D), lambda r: (0, 0)),
        ],
        out_specs=[
            pl.BlockSpec((_BLK, 2 * _D), lambda r: (r, 0)),
            pl.BlockSpec((_BLK, 1), lambda r: (r, 0)),
            pl.BlockSpec((1, _BLK), lambda r: (0, r)),
            pl.BlockSpec((1, _BLK), lambda r: (0, r)),
        ],
        out_shape=[
            jax.ShapeDtypeStruct((_NP, 2 * _D), jnp.bfloat16),
            jax.ShapeDtypeStruct((_NP, 1), jnp.float32),
            jax.ShapeDtypeStruct((1, _NP), jnp.float32),
            jax.ShapeDtypeStruct((1, _NP), jnp.float32),
        ],
        interpret=interpret,
    )(x, w, a_src.reshape(1, _D), a_dst.reshape(1, _D))

    out_specs = [pl.BlockSpec((_BLK, _D), lambda i, j: (i, 0))]
    out_shape = [jax.ShapeDtypeStruct((_NP, _D), jnp.float32)]
    if emit_mask:
        out_specs.append(pl.BlockSpec((_BLK, _BLK), lambda i, j: (i, j)))
        out_shape.append(jax.ShapeDtypeStruct((_N, _N), jnp.int8))
    outs = pl.pallas_call(
        functools.partial(_flash_kernel, emit_mask),
        grid=(_NB, _NB),
        in_specs=[
            pl.BlockSpec((_BLK, _BLK), lambda i, j: (i, j)),
            pl.BlockSpec((_BLK, 1), lambda i, j: (i, 0)),
            pl.BlockSpec((1, _BLK), lambda i, j: (0, j)),
            pl.BlockSpec((1, _BLK), lambda i, j: (0, j)),
            pl.BlockSpec((_BLK, 2 * _D), lambda i, j: (j, 0)),
        ],
        out_specs=out_specs,
        out_shape=out_shape,
        scratch_shapes=[
            pltpu.VMEM((_BLK, 2 * _D), jnp.float32),
        ],
        interpret=interpret,
    )(adj, ee, bb, dd, hext)
    return outs


def kernel(embeddings, adjacency_matrix, W0, a_src0, a_dst0,
           W1, a_src1, a_dst1, interpret=False):
    x, mask8 = _gat_layer(embeddings, adjacency_matrix, W0, a_src0, a_dst0,
                          emit_mask=True, interpret=interpret)
    x, = _gat_layer(x, mask8, W1, a_src1, a_dst1,
                    emit_mask=False, interpret=interpret)
    return x[:_N]
